# baseline (speedup 1.0000x reference)
import functools

import jax
import jax.numpy as jnp
from jax import lax
from jax.experimental import pallas as pl
from jax.experimental.pallas import tpu as pltpu
from jax.experimental.pallas import tpu_sc as plsc

N = 10000
E = 320000
D = 128
ED = 16

NC = 2
NS = 16
NW = NC * NS
VS = 64
SE = E // VS
G = 4
PNG = N // G
FB = 8
GCH = 128
CAPF = 5248

_i32 = jnp.int32
_f32 = jnp.float32


def _iota16():
    return lax.iota(_i32, 16)


def _edge_mm_body(ea_ref, w0_ref, w1_ref, b0_ref, b1_ref):
    ea = ea_ref[...]
    b0_ref[...] = jnp.dot(ea, w0_ref[...], preferred_element_type=_f32)
    b1_ref[...] = jnp.dot(ea, w1_ref[...], preferred_element_type=_f32)


def _edge_mm(ea, we0, we1):
    blk = 2560
    return pl.pallas_call(
        _edge_mm_body,
        grid=(E // blk,),
        in_specs=[
            pl.BlockSpec((blk, ED), lambda i: (i, 0)),
            pl.BlockSpec((ED, D), lambda i: (0, 0)),
            pl.BlockSpec((ED, D), lambda i: (0, 0)),
        ],
        out_specs=[
            pl.BlockSpec((blk, D), lambda i: (i, 0)),
            pl.BlockSpec((blk, D), lambda i: (i, 0)),
        ],
        out_shape=[jax.ShapeDtypeStruct((E, D), _f32)] * 2,
    )(ea, we0, we1)


def _gru_body(emit_hp, x_ref, a_ref, wz, uz, bz, wr, ur, br, wh, uh, bh,
              wg, out_ref):
    x = x_ref[...]
    a = a_ref[...]

    def mm(m, w):
        return jnp.dot(m, w[...], preferred_element_type=_f32)

    z = 1.0 / (1.0 + jnp.exp(-(mm(x, wz) + mm(a, uz) + bz[...])))
    r = 1.0 / (1.0 + jnp.exp(-(mm(x, wr) + mm(a, ur) + br[...])))
    n = jnp.tanh(mm(x, wh) + mm(r * a, uh) + bh[...])
    h = (1.0 - z) * a + z * n
    if emit_hp:
        out_ref[...] = mm(h, wg)
    else:
        out_ref[...] = h


def _gru(x, agg, wz, uz, bz, wr, ur, br, wh, uh, bh, wg=None):
    blk = 2000
    emit_hp = wg is not None
    if wg is None:
        wg = jnp.zeros((D, D), _f32)
    wspec = pl.BlockSpec((D, D), lambda i: (0, 0))
    bspec = pl.BlockSpec((1, D), lambda i: (0, 0))
    return pl.pallas_call(
        functools.partial(_gru_body, emit_hp),
        grid=(N // blk,),
        in_specs=[
            pl.BlockSpec((blk, D), lambda i: (i, 0)),
            pl.BlockSpec((blk, D), lambda i: (i, 0)),
            wspec, wspec, bspec, wspec, wspec, bspec, wspec, wspec, bspec,
            wspec,
        ],
        out_specs=pl.BlockSpec((blk, D), lambda i: (i, 0)),
        out_shape=jax.ShapeDtypeStruct((N, D), _f32),
    )(x, agg, wz, uz, bz, wr, ur, br, wh, uh, bh, wg)


def _part_body(dst_h, src_h, feid, fsrc, fdl, fcnt,
               dbuf, sbuf, stg_e, stg_s, stg_d, cbuf):
    wid = lax.axis_index("s") * NC + lax.axis_index("c")
    iota = _iota16()

    for half in range(2):
        v = wid * 2 + half
        base = v * SE
        pltpu.sync_copy(dst_h.at[pl.ds(base, SE)], dbuf.at[pl.ds(0, SE)])
        pltpu.sync_copy(src_h.at[pl.ds(base, SE)], sbuf.at[pl.ds(0, SE)])
        dbuf[pl.ds(SE, 16)] = jnp.full((16,), 1 << 30, _i32)

        def scan_step(i, ps):
            d = dbuf[pl.ds(i * 16, 16)]
            s = sbuf[pl.ds(i * 16, 16)]
            eidv = base + i * 16 + iota
            new_ps = []
            for g in range(G):
                lo = g * PNG
                msk = (d >= lo) & (d < lo + PNG)
                dl = d - lo
                p = ps[g]
                csum = jnp.cumsum(jnp.where(msk, 1, 0).astype(_i32))
                pos = g * CAPF + p + csum - 1
                plsc.store_scatter(stg_e, [pos], eidv, mask=msk)
                plsc.store_scatter(stg_s, [pos], s, mask=msk)
                plsc.store_scatter(stg_d, [pos], dl, mask=msk)
                new_ps.append(p + jnp.max(csum))
            return tuple(new_ps)

        ps = lax.fori_loop(0, (SE + 15) // 16, scan_step, (0, 0, 0, 0))

        dummy_e = jnp.full((16,), v * 1000, _i32)
        dummy_s = jnp.full((16,), v * 100, _i32)
        dummy_d = jnp.full((16,), PNG, _i32)
        for g in range(G):
            p = ps[g]
            for k in range(GCH // 16):
                stg_e[pl.ds(g * CAPF + p + k * 16, 16)] = dummy_e
                stg_s[pl.ds(g * CAPF + p + k * 16, 16)] = dummy_s
                stg_d[pl.ds(g * CAPF + p + k * 16, 16)] = dummy_d
            nch = ((p + GCH - 1) // GCH).astype(_i32)
            cbuf[pl.ds(g * 128, 16)] = jnp.full((16,), nch, _i32)
            pltpu.sync_copy(stg_e.at[pl.ds(g * CAPF, CAPF)], feid.at[v, g])
            pltpu.sync_copy(stg_s.at[pl.ds(g * CAPF, CAPF)], fsrc.at[v, g])
            pltpu.sync_copy(stg_d.at[pl.ds(g * CAPF, CAPF)], fdl.at[v, g])
        pltpu.sync_copy(cbuf, fcnt.at[v])


def _partition(dst, src):
    mesh = plsc.VectorSubcoreMesh(core_axis_name="c", subcore_axis_name="s")
    k = functools.partial(
        pl.kernel,
        out_type=[
            jax.ShapeDtypeStruct((VS, G, CAPF), _i32),
            jax.ShapeDtypeStruct((VS, G, CAPF), _i32),
            jax.ShapeDtypeStruct((VS, G, CAPF), _i32),
            jax.ShapeDtypeStruct((VS, G * 128), _i32),
        ],
        mesh=mesh,
        compiler_params=pltpu.CompilerParams(needs_layout_passes=False),
        scratch_types=[
            pltpu.VMEM((SE + 16,), _i32),
            pltpu.VMEM((SE + 16,), _i32),
            pltpu.VMEM((G * CAPF,), _i32),
            pltpu.VMEM((G * CAPF,), _i32),
            pltpu.VMEM((G * CAPF,), _i32),
            pltpu.VMEM((G * 128,), _i32),
        ],
    )(_part_body)
    return k(dst, src)


def _segmax_body(with_h, args):
    if with_h:
        (b2, h2, feid, fsrc, fdl, fcnt, out,
         cbuf, ebuf, srbuf, dlbuf, ibb, ibh, bbuf, hbuf, agg, semb, semh) = args
    else:
        (b2, feid, fsrc, fdl, fcnt, out,
         cbuf, ebuf, srbuf, dlbuf, ibb, bbuf, agg, semb) = args
    wid = lax.axis_index("s") * NC + lax.axis_index("c")
    g = wid // FB
    fb = wid % FB
    lo = g * PNG
    iota = _iota16()
    zeros = jnp.zeros((16,), _f32)

    def zstep(i, _):
        plsc.store_scatter(agg, [jnp.full((16,), i, _i32), iota], zeros)
        return 0

    lax.fori_loop(0, PNG + 1, zstep, 0)

    pltpu.sync_copy(fcnt.at[:, pl.ds(g * 128, 128)], cbuf)

    def frag_step(v, _):
        nch = jnp.max(plsc.load_gather(cbuf, [jnp.full((16,), v, _i32), iota]))

        def chunk_step(ch, _c):
            boff = ch * GCH
            pltpu.sync_copy(feid.at[v, g, pl.ds(boff, GCH)], ebuf)
            pltpu.sync_copy(fdl.at[v, g, pl.ds(boff, GCH)], dlbuf)
            if with_h:
                pltpu.sync_copy(fsrc.at[v, g, pl.ds(boff, GCH)], srbuf)
            for j in range(GCH // 16):
                e = ebuf[pl.ds(j * 16, 16)]
                ibb[pl.ds(j * 16, 16)] = e * 8 + fb
                if with_h:
                    s = srbuf[pl.ds(j * 16, 16)]
                    ibh[pl.ds(j * 16, 16)] = s * 8 + fb
            cb = pltpu.async_copy(b2.at[ibb], bbuf, semb)
            if with_h:
                chh = pltpu.async_copy(h2.at[ibh], hbuf, semh)
            cb.wait()
            if with_h:
                chh.wait()

            def hot(k, _h):
                for u in range(8):
                    i = k * 8 + u
                    ispl = jnp.full((16,), i, _i32)
                    dspl = plsc.load_gather(dlbuf, [ispl])
                    m = plsc.load_gather(bbuf, [ispl, iota])
                    if with_h:
                        m = m + plsc.load_gather(hbuf, [ispl, iota])
                    a = plsc.load_gather(agg, [dspl, iota])
                    plsc.store_scatter(agg, [dspl, iota], jnp.maximum(a, m))
                return 0

            lax.fori_loop(0, GCH // 8, hot, 0)
            return 0

        lax.fori_loop(0, nch, chunk_step, 0)
        return 0

    lax.fori_loop(0, VS, frag_step, 0)

    pltpu.sync_copy(agg.at[pl.ds(0, PNG), :], out.at[g, fb])


def _segmax(b2, parts, h2=None):
    feid, fsrc, fdl, fcnt = parts
    with_h = h2 is not None
    mesh = plsc.VectorSubcoreMesh(core_axis_name="c", subcore_axis_name="s")
    scratch = [
        pltpu.VMEM((VS, 128), _i32),
        pltpu.VMEM((GCH,), _i32),
        pltpu.VMEM((GCH,), _i32),
        pltpu.VMEM((GCH,), _i32),
        pltpu.VMEM((GCH,), _i32),
    ]
    if with_h:
        scratch.append(pltpu.VMEM((GCH,), _i32))
    scratch.append(pltpu.VMEM((GCH, 16), _f32))
    if with_h:
        scratch.append(pltpu.VMEM((GCH, 16), _f32))
    scratch.append(pltpu.VMEM((PNG + 1, 16), _f32))
    scratch.append(pltpu.SemaphoreType.DMA)
    if with_h:
        scratch.append(pltpu.SemaphoreType.DMA)

    def body(*args):
        _segmax_body(with_h, args)

    k = functools.partial(
        pl.kernel,
        out_type=jax.ShapeDtypeStruct((G, FB, PNG, 16), _f32),
        mesh=mesh,
        compiler_params=pltpu.CompilerParams(needs_layout_passes=False,
                                             use_tc_tiling_on_sc=False),
        scratch_types=scratch,
    )(body)
    if with_h:
        out4 = k(b2, h2, feid, fsrc, fdl, fcnt)
    else:
        out4 = k(b2, feid, fsrc, fdl, fcnt)
    return jnp.transpose(out4, (0, 2, 1, 3)).reshape(N, D)


def kernel(x, edge_index, edge_attr, Wg, We, Wz, Uz, bz, Wr, Ur, br,
           Wh, Uh, bh):
    src = edge_index[0]
    dst = edge_index[1]

    b0, b1 = _edge_mm(edge_attr, We[0], We[1])
    parts = _partition(dst, src)

    agg0 = _segmax(b0.reshape(E * 8, 16), parts)
    hp1 = _gru(x, agg0, Wz[0], Uz[0], bz[0][None], Wr[0], Ur[0], br[0][None],
               Wh[0], Uh[0], bh[0][None], wg=Wg[1])
    agg1 = _segmax(b1.reshape(E * 8, 16), parts, h2=hp1.reshape(N * 8, 16))
    return _gru(x, agg1, Wz[1], Uz[1], bz[1][None], Wr[1], Ur[1], br[1][None],
                Wh[1], Uh[1], bh[1][None])
